# Initial kernel scaffold; baseline (speedup 1.0000x reference)
import functools

import jax
import jax.numpy as jnp
from jax import lax
from jax.experimental import pallas as pl
from jax.experimental.pallas import tpu as pltpu
from jax.experimental.pallas import tpu_sc as plsc

N = 10000
E = 320000
D_IN = 128
H1 = 256
H2 = 32
C = 16

NC, NS = 2, 16
NW = NC * NS
CH = 128
K = -(-E // (NW * CH))
EP = NW * K * CH
NP = 10240
RPT = NP // NS
RB = 1000


def _make_agg(w):
  mesh = plsc.VectorSubcoreMesh(
      core_axis_name="c", subcore_axis_name="s",
      num_cores=NC, num_subcores=NS)

  @functools.partial(
      pl.kernel,
      out_type=jax.ShapeDtypeStruct((NC, NP, w), jnp.float32),
      mesh=mesh,
      scratch_types=[
          pltpu.VMEM((K, CH), jnp.int32),
          pltpu.VMEM((K, CH), jnp.int32),
          pltpu.VMEM((CH, w), jnp.float32),
          pltpu.VMEM_SHARED((NP, w), jnp.float32),
          pltpu.SemaphoreType.DMA,
      ],
  )
  def agg(g_hbm, src_hbm, dst_hbm, z_hbm, out_hbm, src_v, dst_v, rows_v,
          acc, sem):
    c = lax.axis_index("c")
    s = lax.axis_index("s")
    wid = s * NC + c
    pltpu.sync_copy(z_hbm.at[pl.ds(s * RPT, RPT)], acc.at[pl.ds(s * RPT, RPT)])
    pltpu.sync_copy(src_hbm.at[pl.ds(wid * K, K)], src_v)
    pltpu.sync_copy(dst_hbm.at[pl.ds(wid * K, K)], dst_v)
    plsc.subcore_barrier()

    def body(j, carry):
      pltpu.async_copy(g_hbm.at[src_v.at[j]], rows_v, sem).wait()
      pltpu.sync_copy(rows_v, acc.at[dst_v.at[j]], add=True)
      return carry

    lax.fori_loop(0, K, body, 0)
    plsc.subcore_barrier()
    pltpu.sync_copy(acc.at[pl.ds(s * RPT, RPT)],
                    out_hbm.at[c].at[pl.ds(s * RPT, RPT)])

  return agg


_agg128 = _make_agg(D_IN)
_agg32 = _make_agg(H2)
_agg16 = _make_agg(C)


def _tcA_body(degp_ref, x_ref, dinv_ref, g0_ref):
  deg = degp_ref[0][:, 0:1] + degp_ref[1][:, 0:1] + 1.0
  dinv = lax.rsqrt(deg)
  dinv_ref[...] = dinv
  g0_ref[...] = x_ref[...] * dinv


def _tcB_body(p_ref, g0_ref, dinv_ref, w1_ref, b1_ref, w2_ref, g2_ref):
  dinv = dinv_ref[...]
  a1 = dinv * (p_ref[0] + p_ref[1] + g0_ref[...])
  h1 = jnp.dot(a1, w1_ref[...], preferred_element_type=jnp.float32)
  h1 = jnp.maximum(h1 + b1_ref[...], 0.0)
  g2_ref[...] = dinv * jnp.dot(h1, w2_ref[...],
                               preferred_element_type=jnp.float32)


def _tcC_body(p_ref, g2_ref, dinv_ref, b2_ref, w3_ref, g3_ref):
  dinv = dinv_ref[...]
  a2 = dinv * (p_ref[0] + p_ref[1] + g2_ref[...])
  h2 = jnp.maximum(a2 + b2_ref[...], 0.0)
  g3_ref[...] = dinv * jnp.dot(h2, w3_ref[...],
                               preferred_element_type=jnp.float32)


def _tcD_body(p_ref, g3_ref, dinv_ref, b3_ref, out_ref):
  dinv = dinv_ref[...]
  out_ref[...] = dinv * (p_ref[0] + p_ref[1] + g3_ref[...]) + b3_ref[...]


def _row_spec(w):
  return pl.BlockSpec((RB, w), lambda b: (b, 0))


def _p_spec(w):
  return pl.BlockSpec((NC, RB, w), lambda b: (0, b, 0))


def _full_spec(r, w):
  return pl.BlockSpec((r, w), lambda b: (0, 0))


def _tcA(degp, x):
  return pl.pallas_call(
      _tcA_body,
      grid=(N // RB,),
      in_specs=[_p_spec(C), _row_spec(D_IN)],
      out_specs=[_row_spec(1), _row_spec(D_IN)],
      out_shape=[jax.ShapeDtypeStruct((N, 1), jnp.float32),
                 jax.ShapeDtypeStruct((N, D_IN), jnp.float32)],
  )(degp, x)


def _tcB(p1, g0, dinv, W1, b1, W2):
  return pl.pallas_call(
      _tcB_body,
      grid=(N // RB,),
      in_specs=[_p_spec(D_IN), _row_spec(D_IN), _row_spec(1),
                _full_spec(D_IN, H1), _full_spec(1, H1), _full_spec(H1, H2)],
      out_specs=_row_spec(H2),
      out_shape=jax.ShapeDtypeStruct((N, H2), jnp.float32),
  )(p1, g0, dinv, W1, b1.reshape(1, H1), W2)


def _tcC(p2, g2, dinv, b2, W3):
  return pl.pallas_call(
      _tcC_body,
      grid=(N // RB,),
      in_specs=[_p_spec(H2), _row_spec(H2), _row_spec(1),
                _full_spec(1, H2), _full_spec(H2, C)],
      out_specs=_row_spec(C),
      out_shape=jax.ShapeDtypeStruct((N, C), jnp.float32),
  )(p2, g2, dinv, b2.reshape(1, H2), W3)


def _tcD(p3, g3, dinv, b3):
  return pl.pallas_call(
      _tcD_body,
      grid=(N // RB,),
      in_specs=[_p_spec(C), _row_spec(C), _row_spec(1), _full_spec(1, C)],
      out_specs=_row_spec(C),
      out_shape=jax.ShapeDtypeStruct((N, C), jnp.float32),
  )(p3, g3, dinv, b3.reshape(1, C))


def kernel(x, edge_index, W1, b1, W2, b2, W3, b3):
  src = edge_index[0].astype(jnp.int32)
  dst = edge_index[1].astype(jnp.int32)
  pad = EP - E
  src_r = jnp.concatenate([src, jnp.zeros((pad,), jnp.int32)]).reshape(
      NW * K, CH)
  dst_r = jnp.concatenate([dst, jnp.full((pad,), N, jnp.int32)]).reshape(
      NW * K, CH)

  z16 = jnp.zeros((NP, C), jnp.float32)
  z32 = jnp.zeros((NP, H2), jnp.float32)
  z128 = jnp.zeros((NP, D_IN), jnp.float32)
  ones = jnp.ones((N, C), jnp.float32)

  degp = _agg16(ones, src_r, dst_r, z16)
  dinv, g0 = _tcA(degp, x)
  p1 = _agg128(g0, src_r, dst_r, z128)
  g2 = _tcB(p1, g0, dinv, W1, b1, W2)
  p2 = _agg32(g2, src_r, dst_r, z32)
  g3 = _tcC(p2, g2, dinv, b2, W3)
  p3 = _agg16(g3, src_r, dst_r, z16)
  return _tcD(p3, g3, dinv, b3)


# speedup vs baseline: 12.1359x; 12.1359x over previous
import functools

import jax
import jax.numpy as jnp
from jax import lax
from jax.experimental import pallas as pl
from jax.experimental.pallas import tpu as pltpu
from jax.experimental.pallas import tpu_sc as plsc

N = 10000
E = 320000
D_IN = 128
H1 = 256
H2 = 32
C = 16

NC, NS = 2, 16
NW = NC * NS
CH = 128
K = 80
EP = NW * K * CH
NP = 10240
RPT = NP // NS
RB = 1000


def _make_agg(w):
  mesh = plsc.VectorSubcoreMesh(
      core_axis_name="c", subcore_axis_name="s",
      num_cores=NC, num_subcores=NS)

  @functools.partial(
      pl.kernel,
      out_type=jax.ShapeDtypeStruct((NC, NP, w), jnp.float32),
      mesh=mesh,
      compiler_params=pltpu.CompilerParams(use_tc_tiling_on_sc=False),
      scratch_types=[
          pltpu.VMEM((K, CH), jnp.int32),
          pltpu.VMEM((K, CH), jnp.int32),
          pltpu.VMEM((CH, w), jnp.float32),
          pltpu.VMEM_SHARED((NP, w), jnp.float32),
          pltpu.SemaphoreType.DMA,
      ],
  )
  def agg(g_hbm, src_hbm, dst_hbm, z_hbm, out_hbm, src_v, dst_v, rows_v,
          acc, sem):
    c = lax.axis_index("c")
    s = lax.axis_index("s")
    wid = s * NC + c
    pltpu.sync_copy(z_hbm.at[pl.ds(s * RPT, RPT)], acc.at[pl.ds(s * RPT, RPT)])
    pltpu.sync_copy(src_hbm.at[pl.ds(wid * K, K)], src_v)
    pltpu.sync_copy(dst_hbm.at[pl.ds(wid * K, K)], dst_v)
    plsc.subcore_barrier()

    def body(j, carry):
      pltpu.async_copy(g_hbm.at[src_v.at[j]], rows_v, sem).wait()
      pltpu.sync_copy(rows_v, acc.at[dst_v.at[j]], add=True)
      return carry

    lax.fori_loop(0, K, body, 0)
    plsc.subcore_barrier()
    pltpu.sync_copy(acc.at[pl.ds(s * RPT, RPT)],
                    out_hbm.at[c].at[pl.ds(s * RPT, RPT)])

  return agg


_agg128 = _make_agg(D_IN)
_agg32 = _make_agg(H2)
_agg16 = _make_agg(C)


def _tcA_body(degp_ref, x_ref, dinv_ref, g0_ref):
  deg = degp_ref[0][:, 0:1] + degp_ref[1][:, 0:1] + 1.0
  dinv = lax.rsqrt(deg)
  dinv_ref[...] = dinv
  g0_ref[...] = x_ref[...] * dinv


def _tcB_body(p_ref, g0_ref, dinv_ref, w1_ref, b1_ref, w2_ref, g2_ref):
  dinv = dinv_ref[...]
  a1 = dinv * (p_ref[0] + p_ref[1] + g0_ref[...])
  h1 = jnp.dot(a1, w1_ref[...], preferred_element_type=jnp.float32)
  h1 = jnp.maximum(h1 + b1_ref[...], 0.0)
  g2_ref[...] = dinv * jnp.dot(h1, w2_ref[...],
                               preferred_element_type=jnp.float32)


def _tcC_body(p_ref, g2_ref, dinv_ref, b2_ref, w3_ref, g3_ref):
  dinv = dinv_ref[...]
  a2 = dinv * (p_ref[0] + p_ref[1] + g2_ref[...])
  h2 = jnp.maximum(a2 + b2_ref[...], 0.0)
  g3_ref[...] = dinv * jnp.dot(h2, w3_ref[...],
                               preferred_element_type=jnp.float32)


def _tcD_body(p_ref, g3_ref, dinv_ref, b3_ref, out_ref):
  dinv = dinv_ref[...]
  out_ref[...] = dinv * (p_ref[0] + p_ref[1] + g3_ref[...]) + b3_ref[...]


def _row_spec(w):
  return pl.BlockSpec((RB, w), lambda b: (b, 0))


def _p_spec(w):
  return pl.BlockSpec((NC, RB, w), lambda b: (0, b, 0))


def _full_spec(r, w):
  return pl.BlockSpec((r, w), lambda b: (0, 0))


def _tcA(degp, x):
  return pl.pallas_call(
      _tcA_body,
      grid=(N // RB,),
      in_specs=[_p_spec(C), _row_spec(D_IN)],
      out_specs=[_row_spec(1), _row_spec(D_IN)],
      out_shape=[jax.ShapeDtypeStruct((N, 1), jnp.float32),
                 jax.ShapeDtypeStruct((N, D_IN), jnp.float32)],
  )(degp, x)


def _tcB(p1, g0, dinv, W1, b1, W2):
  return pl.pallas_call(
      _tcB_body,
      grid=(N // RB,),
      in_specs=[_p_spec(D_IN), _row_spec(D_IN), _row_spec(1),
                _full_spec(D_IN, H1), _full_spec(1, H1), _full_spec(H1, H2)],
      out_specs=_row_spec(H2),
      out_shape=jax.ShapeDtypeStruct((N, H2), jnp.float32),
  )(p1, g0, dinv, W1, b1.reshape(1, H1), W2)


def _tcC(p2, g2, dinv, b2, W3):
  return pl.pallas_call(
      _tcC_body,
      grid=(N // RB,),
      in_specs=[_p_spec(H2), _row_spec(H2), _row_spec(1),
                _full_spec(1, H2), _full_spec(H2, C)],
      out_specs=_row_spec(C),
      out_shape=jax.ShapeDtypeStruct((N, C), jnp.float32),
  )(p2, g2, dinv, b2.reshape(1, H2), W3)


def _tcD(p3, g3, dinv, b3):
  return pl.pallas_call(
      _tcD_body,
      grid=(N // RB,),
      in_specs=[_p_spec(C), _row_spec(C), _row_spec(1), _full_spec(1, C)],
      out_specs=_row_spec(C),
      out_shape=jax.ShapeDtypeStruct((N, C), jnp.float32),
  )(p3, g3, dinv, b3.reshape(1, C))


def kernel(x, edge_index, W1, b1, W2, b2, W3, b3):
  src = edge_index[0].astype(jnp.int32)
  dst = edge_index[1].astype(jnp.int32)
  pad = EP - E
  src_r = jnp.concatenate([src, jnp.zeros((pad,), jnp.int32)]).reshape(
      NW * K, CH)
  dst_r = jnp.concatenate([dst, jnp.full((pad,), N, jnp.int32)]).reshape(
      NW * K, CH)

  z16 = jnp.zeros((NP, C), jnp.float32)
  z32 = jnp.zeros((NP, H2), jnp.float32)
  z128 = jnp.zeros((NP, D_IN), jnp.float32)
  ones = jnp.ones((N, C), jnp.float32)

  degp = _agg16(ones, src_r, dst_r, z16)
  dinv, g0 = _tcA(degp, x)
  p1 = _agg128(g0, src_r, dst_r, z128)
  g2 = _tcB(p1, g0, dinv, W1, b1, W2)
  p2 = _agg32(g2, src_r, dst_r, z32)
  g3 = _tcC(p2, g2, dinv, b2, W3)
  p3 = _agg16(g3, src_r, dst_r, z16)
  return _tcD(p3, g3, dinv, b3)


# speedup vs baseline: 20.7523x; 1.7100x over previous
import functools

import jax
import jax.numpy as jnp
from jax import lax
from jax.experimental import pallas as pl
from jax.experimental.pallas import tpu as pltpu
from jax.experimental.pallas import tpu_sc as plsc

N = 10000
E = 320000
D_IN = 128
H1 = 256
H2 = 32
C = 16

NC, NS = 2, 16
NW = NC * NS
CH = 128
K = 80
EP = NW * K * CH
NP = 10240
RPT = NP // NS
RB = 1000


def _make_agg(w, ge):
  ept = K * CH
  ng = ept // ge
  nsc = ge // CH
  assert ng % 2 == 0 and ept % ge == 0
  mesh = plsc.VectorSubcoreMesh(
      core_axis_name="c", subcore_axis_name="s",
      num_cores=NC, num_subcores=NS)

  @functools.partial(
      pl.kernel,
      out_type=jax.ShapeDtypeStruct((NC, NP, w), jnp.float32),
      mesh=mesh,
      compiler_params=pltpu.CompilerParams(use_tc_tiling_on_sc=False),
      scratch_types=[
          pltpu.VMEM((ept,), jnp.int32),
          pltpu.VMEM((K, CH), jnp.int32),
          pltpu.VMEM((2, ge, w), jnp.float32),
          pltpu.VMEM_SHARED((NP, w), jnp.float32),
          pltpu.SemaphoreType.DMA,
          pltpu.SemaphoreType.DMA,
      ],
  )
  def agg(g_hbm, srcf_hbm, dst_hbm, z_hbm, out_hbm, src_v, dst_v, rows_v,
          acc, sem0, sem1):
    sems = (sem0, sem1)
    c = lax.axis_index("c")
    s = lax.axis_index("s")
    wid = s * NC + c
    pltpu.sync_copy(z_hbm.at[pl.ds(s * RPT, RPT)], acc.at[pl.ds(s * RPT, RPT)])
    pltpu.sync_copy(srcf_hbm.at[pl.ds(wid * ept, ept)], src_v)
    pltpu.sync_copy(dst_hbm.at[pl.ds(wid * K, K)], dst_v)
    plsc.subcore_barrier()

    def gstart(j, b):
      pltpu.async_copy(g_hbm.at[src_v.at[pl.ds(j * ge, ge)]],
                       rows_v.at[b], sems[b])

    def gwait(j, b):
      pltpu.make_async_copy(g_hbm.at[src_v.at[pl.ds(j * ge, ge)]],
                            rows_v.at[b], sems[b]).wait()

    def scatter(j, b):
      for q in range(nsc):
        pltpu.sync_copy(rows_v.at[b].at[pl.ds(q * CH, CH)],
                        acc.at[dst_v.at[j * nsc + q]], add=True)

    gstart(0, 0)
    gstart(1, 1)

    def body(jj, carry):
      for b in (0, 1):
        j = 2 * jj + b
        gwait(j, b)
        scatter(j, b)
        gstart(j + 2, b)
      return carry

    lax.fori_loop(0, (ng - 2) // 2, body, 0)
    for b in (0, 1):
      j = ng - 2 + b
      gwait(j, b)
      scatter(j, b)

    plsc.subcore_barrier()
    pltpu.sync_copy(acc.at[pl.ds(s * RPT, RPT)],
                    out_hbm.at[c].at[pl.ds(s * RPT, RPT)])

  return agg


def _make_agg_split(ge=256):
  wh = D_IN // 2
  ept = EP // NS
  k2 = ept // CH
  ng = ept // ge
  nsc = ge // CH
  assert ng % 2 == 0 and ept % ge == 0
  mesh = plsc.VectorSubcoreMesh(
      core_axis_name="c", subcore_axis_name="s",
      num_cores=NC, num_subcores=NS)

  @functools.partial(
      pl.kernel,
      out_type=jax.ShapeDtypeStruct((NC, NP, wh), jnp.float32),
      mesh=mesh,
      compiler_params=pltpu.CompilerParams(use_tc_tiling_on_sc=False),
      scratch_types=[
          pltpu.VMEM((ept,), jnp.int32),
          pltpu.VMEM((k2, CH), jnp.int32),
          pltpu.VMEM((2, ge, wh), jnp.float32),
          pltpu.VMEM_SHARED((NP, wh), jnp.float32),
          pltpu.SemaphoreType.DMA,
          pltpu.SemaphoreType.DMA,
      ],
  )
  def agg(gcat_hbm, src2_hbm, dst_hbm, z_hbm, out_hbm, src_v, dst_v, rows_v,
          acc, sem0, sem1):
    sems = (sem0, sem1)
    c = lax.axis_index("c")
    s = lax.axis_index("s")
    pltpu.sync_copy(z_hbm.at[pl.ds(s * RPT, RPT)], acc.at[pl.ds(s * RPT, RPT)])
    pltpu.sync_copy(src2_hbm.at[c].at[pl.ds(s * ept, ept)], src_v)
    pltpu.sync_copy(dst_hbm.at[pl.ds(s * k2, k2)], dst_v)
    plsc.subcore_barrier()

    def gstart(j, b):
      pltpu.async_copy(gcat_hbm.at[src_v.at[pl.ds(j * ge, ge)]],
                       rows_v.at[b], sems[b])

    def gwait(j, b):
      pltpu.make_async_copy(gcat_hbm.at[src_v.at[pl.ds(j * ge, ge)]],
                            rows_v.at[b], sems[b]).wait()

    def scatter(j, b):
      for q in range(nsc):
        pltpu.sync_copy(rows_v.at[b].at[pl.ds(q * CH, CH)],
                        acc.at[dst_v.at[j * nsc + q]], add=True)

    gstart(0, 0)
    gstart(1, 1)

    def body(jj, carry):
      for b in (0, 1):
        j = 2 * jj + b
        gwait(j, b)
        scatter(j, b)
        gstart(j + 2, b)
      return carry

    lax.fori_loop(0, (ng - 2) // 2, body, 0)
    for b in (0, 1):
      j = ng - 2 + b
      gwait(j, b)
      scatter(j, b)

    plsc.subcore_barrier()
    pltpu.sync_copy(acc.at[pl.ds(s * RPT, RPT)],
                    out_hbm.at[c].at[pl.ds(s * RPT, RPT)])

  return agg


def _make_deg():
  mesh = plsc.VectorSubcoreMesh(
      core_axis_name="c", subcore_axis_name="s",
      num_cores=NC, num_subcores=NS)

  @functools.partial(
      pl.kernel,
      out_type=jax.ShapeDtypeStruct((NC, NP, C), jnp.float32),
      mesh=mesh,
      compiler_params=pltpu.CompilerParams(use_tc_tiling_on_sc=False),
      scratch_types=[
          pltpu.VMEM((K, CH), jnp.int32),
          pltpu.VMEM((CH, C), jnp.float32),
          pltpu.VMEM_SHARED((NP, C), jnp.float32),
      ],
  )
  def deg(ones_hbm, dst_hbm, z_hbm, out_hbm, dst_v, ones_v, acc):
    c = lax.axis_index("c")
    s = lax.axis_index("s")
    wid = s * NC + c
    pltpu.sync_copy(z_hbm.at[pl.ds(s * RPT, RPT)], acc.at[pl.ds(s * RPT, RPT)])
    pltpu.sync_copy(ones_hbm, ones_v)
    pltpu.sync_copy(dst_hbm.at[pl.ds(wid * K, K)], dst_v)
    plsc.subcore_barrier()

    def body(j, carry):
      pltpu.sync_copy(ones_v, acc.at[dst_v.at[j]], add=True)
      return carry

    lax.fori_loop(0, K, body, 0)
    plsc.subcore_barrier()
    pltpu.sync_copy(acc.at[pl.ds(s * RPT, RPT)],
                    out_hbm.at[c].at[pl.ds(s * RPT, RPT)])

  return deg


_agg128 = _make_agg_split(256)
_agg32 = _make_agg(H2, 1024)
_agg16 = _make_agg(C, 1024)
_deg16 = _make_deg()


def _tcA_body(degp_ref, x_ref, dinv_ref, g0cat_ref):
  deg = degp_ref[0][:, 0:1] + degp_ref[1][:, 0:1] + 1.0
  dinv = lax.rsqrt(deg)
  dinv_ref[...] = dinv
  g0 = x_ref[...] * dinv
  g0cat_ref[0] = g0[:, :D_IN // 2]
  g0cat_ref[1] = g0[:, D_IN // 2:]


def _tcB_body(p_ref, g0cat_ref, dinv_ref, w1_ref, b1_ref, w2_ref, g2_ref):
  dinv = dinv_ref[...]
  pcat = jnp.concatenate([p_ref[0], p_ref[1]], axis=1)
  gcat = jnp.concatenate([g0cat_ref[0], g0cat_ref[1]], axis=1)
  a1 = dinv * (pcat + gcat)
  h1 = jnp.dot(a1, w1_ref[...], preferred_element_type=jnp.float32)
  h1 = jnp.maximum(h1 + b1_ref[...], 0.0)
  g2_ref[...] = dinv * jnp.dot(h1, w2_ref[...],
                               preferred_element_type=jnp.float32)


def _tcC_body(p_ref, g2_ref, dinv_ref, b2_ref, w3_ref, g3_ref):
  dinv = dinv_ref[...]
  a2 = dinv * (p_ref[0] + p_ref[1] + g2_ref[...])
  h2 = jnp.maximum(a2 + b2_ref[...], 0.0)
  g3_ref[...] = dinv * jnp.dot(h2, w3_ref[...],
                               preferred_element_type=jnp.float32)


def _tcD_body(p_ref, g3_ref, dinv_ref, b3_ref, out_ref):
  dinv = dinv_ref[...]
  out_ref[...] = dinv * (p_ref[0] + p_ref[1] + g3_ref[...]) + b3_ref[...]


def _row_spec(w):
  return pl.BlockSpec((RB, w), lambda b: (b, 0))


def _p_spec(w):
  return pl.BlockSpec((NC, RB, w), lambda b: (0, b, 0))


def _full_spec(r, w):
  return pl.BlockSpec((r, w), lambda b: (0, 0))


def _tcA(degp, x):
  return pl.pallas_call(
      _tcA_body,
      grid=(N // RB,),
      in_specs=[_p_spec(C), _row_spec(D_IN)],
      out_specs=[_row_spec(1), _p_spec(D_IN // 2)],
      out_shape=[jax.ShapeDtypeStruct((N, 1), jnp.float32),
                 jax.ShapeDtypeStruct((NC, N, D_IN // 2), jnp.float32)],
  )(degp, x)


def _tcB(p1, g0cat, dinv, W1, b1, W2):
  return pl.pallas_call(
      _tcB_body,
      grid=(N // RB,),
      in_specs=[_p_spec(D_IN // 2), _p_spec(D_IN // 2), _row_spec(1),
                _full_spec(D_IN, H1), _full_spec(1, H1), _full_spec(H1, H2)],
      out_specs=_row_spec(H2),
      out_shape=jax.ShapeDtypeStruct((N, H2), jnp.float32),
  )(p1, g0cat, dinv, W1, b1.reshape(1, H1), W2)


def _tcC(p2, g2, dinv, b2, W3):
  return pl.pallas_call(
      _tcC_body,
      grid=(N // RB,),
      in_specs=[_p_spec(H2), _row_spec(H2), _row_spec(1),
                _full_spec(1, H2), _full_spec(H2, C)],
      out_specs=_row_spec(C),
      out_shape=jax.ShapeDtypeStruct((N, C), jnp.float32),
  )(p2, g2, dinv, b2.reshape(1, H2), W3)


def _tcD(p3, g3, dinv, b3):
  return pl.pallas_call(
      _tcD_body,
      grid=(N // RB,),
      in_specs=[_p_spec(C), _row_spec(C), _row_spec(1), _full_spec(1, C)],
      out_specs=_row_spec(C),
      out_shape=jax.ShapeDtypeStruct((N, C), jnp.float32),
  )(p3, g3, dinv, b3.reshape(1, C))


def kernel(x, edge_index, W1, b1, W2, b2, W3, b3):
  src = edge_index[0].astype(jnp.int32)
  dst = edge_index[1].astype(jnp.int32)
  pad = EP - E
  src_r = jnp.concatenate([src, jnp.zeros((pad,), jnp.int32)]).reshape(
      NW * K, CH)
  dst_r = jnp.concatenate([dst, jnp.full((pad,), N, jnp.int32)]).reshape(
      NW * K, CH)

  src_f = src_r.reshape(-1)
  src2 = jnp.stack([src_f, src_f + N])

  z16 = jnp.zeros((NP, C), jnp.float32)
  z32 = jnp.zeros((NP, H2), jnp.float32)
  z64 = jnp.zeros((NP, D_IN // 2), jnp.float32)
  ones = jnp.ones((CH, C), jnp.float32)

  degp = _deg16(ones, dst_r, z16)
  dinv, g0cat = _tcA(degp, x)
  p1 = _agg128(g0cat.reshape(NC * N, D_IN // 2), src2, dst_r, z64)
  g2 = _tcB(p1, g0cat, dinv, W1, b1, W2)
  p2 = _agg32(g2, src_f, dst_r, z32)
  g3 = _tcC(p2, g2, dinv, b2, W3)
  p3 = _agg16(g3, src_f, dst_r, z16)
  return _tcD(p3, g3, dinv, b3)


# speedup vs baseline: 29.1849x; 1.4063x over previous
import functools

import jax
import jax.numpy as jnp
from jax import lax
from jax.experimental import pallas as pl
from jax.experimental.pallas import tpu as pltpu
from jax.experimental.pallas import tpu_sc as plsc

N = 10000
E = 320000
D_IN = 128
H1 = 256
H2 = 32
C = 16

NC, NS = 2, 16
NW = NC * NS
CH = 128
K = 80
EP = NW * K * CH
NP = 10240
RPT = NP // NS
RB = 1000


def _make_agg(w, ge, dt=jnp.float32):
  ept = K * CH
  ng = ept // ge
  nsc = ge // CH
  assert ng % 2 == 0 and ept % ge == 0
  mesh = plsc.VectorSubcoreMesh(
      core_axis_name="c", subcore_axis_name="s",
      num_cores=NC, num_subcores=NS)

  @functools.partial(
      pl.kernel,
      out_type=jax.ShapeDtypeStruct((NC, NP, w), dt),
      mesh=mesh,
      compiler_params=pltpu.CompilerParams(use_tc_tiling_on_sc=False),
      scratch_types=[
          pltpu.VMEM((ept,), jnp.int32),
          pltpu.VMEM((K, CH), jnp.int32),
          pltpu.VMEM((2, ge, w), dt),
          pltpu.VMEM_SHARED((NP, w), dt),
          pltpu.SemaphoreType.DMA,
          pltpu.SemaphoreType.DMA,
      ],
  )
  def agg(g_hbm, srcf_hbm, dst_hbm, z_hbm, out_hbm, src_v, dst_v, rows_v,
          acc, sem0, sem1):
    sems = (sem0, sem1)
    c = lax.axis_index("c")
    s = lax.axis_index("s")
    wid = s * NC + c
    pltpu.sync_copy(z_hbm.at[pl.ds(s * RPT, RPT)], acc.at[pl.ds(s * RPT, RPT)])
    pltpu.sync_copy(srcf_hbm.at[pl.ds(wid * ept, ept)], src_v)
    pltpu.sync_copy(dst_hbm.at[pl.ds(wid * K, K)], dst_v)
    plsc.subcore_barrier()

    def gstart(j, b):
      pltpu.async_copy(g_hbm.at[src_v.at[pl.ds(j * ge, ge)]],
                       rows_v.at[b], sems[b])

    def gwait(j, b):
      pltpu.make_async_copy(g_hbm.at[src_v.at[pl.ds(j * ge, ge)]],
                            rows_v.at[b], sems[b]).wait()

    def scatter(j, b):
      for q in range(nsc):
        pltpu.sync_copy(rows_v.at[b].at[pl.ds(q * CH, CH)],
                        acc.at[dst_v.at[j * nsc + q]], add=True)

    gstart(0, 0)
    gstart(1, 1)

    def body(jj, carry):
      for b in (0, 1):
        j = 2 * jj + b
        gwait(j, b)
        scatter(j, b)
        gstart(j + 2, b)
      return carry

    lax.fori_loop(0, (ng - 2) // 2, body, 0)
    for b in (0, 1):
      j = ng - 2 + b
      gwait(j, b)
      scatter(j, b)

    plsc.subcore_barrier()
    pltpu.sync_copy(acc.at[pl.ds(s * RPT, RPT)],
                    out_hbm.at[c].at[pl.ds(s * RPT, RPT)])

  return agg


def _make_agg_split(ge=512, dt=jnp.bfloat16):
  wh = D_IN // 2
  ept = EP // NS
  k2 = ept // CH
  ng = ept // ge
  nsc = ge // CH
  assert ng % 2 == 0 and ept % ge == 0
  mesh = plsc.VectorSubcoreMesh(
      core_axis_name="c", subcore_axis_name="s",
      num_cores=NC, num_subcores=NS)

  @functools.partial(
      pl.kernel,
      out_type=jax.ShapeDtypeStruct((NC, NP, wh), dt),
      mesh=mesh,
      compiler_params=pltpu.CompilerParams(use_tc_tiling_on_sc=False),
      scratch_types=[
          pltpu.VMEM((ept,), jnp.int32),
          pltpu.VMEM((k2, CH), jnp.int32),
          pltpu.VMEM((2, ge, wh), dt),
          pltpu.VMEM_SHARED((NP, wh), dt),
          pltpu.SemaphoreType.DMA,
          pltpu.SemaphoreType.DMA,
      ],
  )
  def agg(gcat_hbm, src2_hbm, dst_hbm, z_hbm, out_hbm, src_v, dst_v, rows_v,
          acc, sem0, sem1):
    sems = (sem0, sem1)
    c = lax.axis_index("c")
    s = lax.axis_index("s")
    pltpu.sync_copy(z_hbm.at[pl.ds(s * RPT, RPT)], acc.at[pl.ds(s * RPT, RPT)])
    pltpu.sync_copy(src2_hbm.at[c].at[pl.ds(s * ept, ept)], src_v)
    pltpu.sync_copy(dst_hbm.at[pl.ds(s * k2, k2)], dst_v)
    plsc.subcore_barrier()

    def gstart(j, b):
      pltpu.async_copy(gcat_hbm.at[src_v.at[pl.ds(j * ge, ge)]],
                       rows_v.at[b], sems[b])

    def gwait(j, b):
      pltpu.make_async_copy(gcat_hbm.at[src_v.at[pl.ds(j * ge, ge)]],
                            rows_v.at[b], sems[b]).wait()

    def scatter(j, b):
      for q in range(nsc):
        pltpu.sync_copy(rows_v.at[b].at[pl.ds(q * CH, CH)],
                        acc.at[dst_v.at[j * nsc + q]], add=True)

    gstart(0, 0)
    gstart(1, 1)

    def body(jj, carry):
      for b in (0, 1):
        j = 2 * jj + b
        gwait(j, b)
        scatter(j, b)
        gstart(j + 2, b)
      return carry

    lax.fori_loop(0, (ng - 2) // 2, body, 0)
    for b in (0, 1):
      j = ng - 2 + b
      gwait(j, b)
      scatter(j, b)

    plsc.subcore_barrier()
    pltpu.sync_copy(acc.at[pl.ds(s * RPT, RPT)],
                    out_hbm.at[c].at[pl.ds(s * RPT, RPT)])

  return agg


def _make_deg():
  mesh = plsc.VectorSubcoreMesh(
      core_axis_name="c", subcore_axis_name="s",
      num_cores=NC, num_subcores=NS)

  @functools.partial(
      pl.kernel,
      out_type=jax.ShapeDtypeStruct((NC, NP, C), jnp.float32),
      mesh=mesh,
      compiler_params=pltpu.CompilerParams(use_tc_tiling_on_sc=False),
      scratch_types=[
          pltpu.VMEM((K, CH), jnp.int32),
          pltpu.VMEM((CH, C), jnp.float32),
          pltpu.VMEM_SHARED((NP, C), jnp.float32),
      ],
  )
  def deg(ones_hbm, dst_hbm, z_hbm, out_hbm, dst_v, ones_v, acc):
    c = lax.axis_index("c")
    s = lax.axis_index("s")
    wid = s * NC + c
    pltpu.sync_copy(z_hbm.at[pl.ds(s * RPT, RPT)], acc.at[pl.ds(s * RPT, RPT)])
    pltpu.sync_copy(ones_hbm, ones_v)
    pltpu.sync_copy(dst_hbm.at[pl.ds(wid * K, K)], dst_v)
    plsc.subcore_barrier()

    def body(j, carry):
      pltpu.sync_copy(ones_v, acc.at[dst_v.at[j]], add=True)
      return carry

    lax.fori_loop(0, K, body, 0)
    plsc.subcore_barrier()
    pltpu.sync_copy(acc.at[pl.ds(s * RPT, RPT)],
                    out_hbm.at[c].at[pl.ds(s * RPT, RPT)])

  return deg


_agg128 = _make_agg_split(512, jnp.bfloat16)
_agg32 = _make_agg(H2, 1024, jnp.bfloat16)
_agg16 = _make_agg(C, 1024, jnp.float32)
_deg16 = _make_deg()


def _tcA_body(degp_ref, x_ref, dinv_ref, g0cat_ref):
  deg = degp_ref[0][:, 0:1] + degp_ref[1][:, 0:1] + 1.0
  dinv = lax.rsqrt(deg)
  dinv_ref[...] = dinv
  g0 = x_ref[...] * dinv
  g0cat_ref[0] = g0[:, :D_IN // 2].astype(jnp.bfloat16)
  g0cat_ref[1] = g0[:, D_IN // 2:].astype(jnp.bfloat16)


def _tcB_body(p_ref, g0cat_ref, dinv_ref, w1_ref, b1_ref, w2_ref, g2_ref):
  dinv = dinv_ref[...]
  pcat = jnp.concatenate([p_ref[0], p_ref[1]], axis=1).astype(jnp.float32)
  gcat = jnp.concatenate([g0cat_ref[0], g0cat_ref[1]],
                         axis=1).astype(jnp.float32)
  a1 = dinv * (pcat + gcat)
  h1 = jnp.dot(a1, w1_ref[...], preferred_element_type=jnp.float32)
  h1 = jnp.maximum(h1 + b1_ref[...], 0.0)
  g2 = dinv * jnp.dot(h1, w2_ref[...], preferred_element_type=jnp.float32)
  g2_ref[...] = g2.astype(jnp.bfloat16)


def _tcC_body(p_ref, g2_ref, dinv_ref, b2_ref, w3_ref, g3_ref):
  dinv = dinv_ref[...]
  p01 = (p_ref[0] + p_ref[1]).astype(jnp.float32)
  a2 = dinv * (p01 + g2_ref[...].astype(jnp.float32))
  h2 = jnp.maximum(a2 + b2_ref[...], 0.0)
  g3_ref[...] = dinv * jnp.dot(h2, w3_ref[...],
                               preferred_element_type=jnp.float32)


def _tcD_body(p_ref, g3_ref, dinv_ref, b3_ref, out_ref):
  dinv = dinv_ref[...]
  out_ref[...] = dinv * (p_ref[0] + p_ref[1] + g3_ref[...]) + b3_ref[...]


def _row_spec(w):
  return pl.BlockSpec((RB, w), lambda b: (b, 0))


def _p_spec(w):
  return pl.BlockSpec((NC, RB, w), lambda b: (0, b, 0))


def _full_spec(r, w):
  return pl.BlockSpec((r, w), lambda b: (0, 0))


def _tcA(degp, x):
  return pl.pallas_call(
      _tcA_body,
      grid=(N // RB,),
      in_specs=[_p_spec(C), _row_spec(D_IN)],
      out_specs=[_row_spec(1), _p_spec(D_IN // 2)],
      out_shape=[jax.ShapeDtypeStruct((N, 1), jnp.float32),
                 jax.ShapeDtypeStruct((NC, N, D_IN // 2), jnp.bfloat16)],
  )(degp, x)


def _tcB(p1, g0cat, dinv, W1, b1, W2):
  return pl.pallas_call(
      _tcB_body,
      grid=(N // RB,),
      in_specs=[_p_spec(D_IN // 2), _p_spec(D_IN // 2), _row_spec(1),
                _full_spec(D_IN, H1), _full_spec(1, H1), _full_spec(H1, H2)],
      out_specs=_row_spec(H2),
      out_shape=jax.ShapeDtypeStruct((N, H2), jnp.bfloat16),
  )(p1, g0cat, dinv, W1, b1.reshape(1, H1), W2)


def _tcC(p2, g2, dinv, b2, W3):
  return pl.pallas_call(
      _tcC_body,
      grid=(N // RB,),
      in_specs=[_p_spec(H2), _row_spec(H2), _row_spec(1),
                _full_spec(1, H2), _full_spec(H2, C)],
      out_specs=_row_spec(C),
      out_shape=jax.ShapeDtypeStruct((N, C), jnp.float32),
  )(p2, g2, dinv, b2.reshape(1, H2), W3)


def _tcD(p3, g3, dinv, b3):
  return pl.pallas_call(
      _tcD_body,
      grid=(N // RB,),
      in_specs=[_p_spec(C), _row_spec(C), _row_spec(1), _full_spec(1, C)],
      out_specs=_row_spec(C),
      out_shape=jax.ShapeDtypeStruct((N, C), jnp.float32),
  )(p3, g3, dinv, b3.reshape(1, C))


def kernel(x, edge_index, W1, b1, W2, b2, W3, b3):
  src = edge_index[0].astype(jnp.int32)
  dst = edge_index[1].astype(jnp.int32)
  pad = EP - E
  src_r = jnp.concatenate([src, jnp.zeros((pad,), jnp.int32)]).reshape(
      NW * K, CH)
  dst_r = jnp.concatenate([dst, jnp.full((pad,), N, jnp.int32)]).reshape(
      NW * K, CH)

  src_f = src_r.reshape(-1)
  src2 = jnp.stack([src_f, src_f + N])

  z16 = jnp.zeros((NP, C), jnp.float32)
  z32 = jnp.zeros((NP, H2), jnp.bfloat16)
  z64 = jnp.zeros((NP, D_IN // 2), jnp.bfloat16)
  ones = jnp.ones((CH, C), jnp.float32)

  degp = _deg16(ones, dst_r, z16)
  dinv, g0cat = _tcA(degp, x)
  p1 = _agg128(g0cat.reshape(NC * N, D_IN // 2), src2, dst_r, z64)
  g2 = _tcB(p1, g0cat, dinv, W1, b1, W2)
  p2 = _agg32(g2, src_f, dst_r, z32)
  g3 = _tcC(p2, g2, dinv, b2, W3)
  p3 = _agg16(g3, src_f, dst_r, z16)
  return _tcD(p3, g3, dinv, b3)
